# baseline (speedup 1.0000x reference)
import jax
import jax.numpy as jnp
from jax import lax
from jax.experimental import pallas as pl
from jax.experimental.pallas import tpu as pltpu
from jax.experimental.pallas import tpu_sc as plsc

HD = 64
NODE_TYPES = ['Depot', 'Satellite', 'Customer', 'RechargingStation']
RELS = [('cc_near', 'Customer', 'Customer'), ('cs_near', 'Customer', 'Satellite'), ('sc_serves', 'Satellite', 'Customer'), ('cc_follows', 'Customer', 'Customer'), ('dr_near', 'Depot', 'RechargingStation'), ('sr_near', 'Satellite', 'RechargingStation'), ('cr_near', 'Customer', 'RechargingStation'), ('rr_near', 'RechargingStation', 'RechargingStation')]
NN = {'Depot': 1, 'Satellite': 1000, 'Customer': 100000, 'RechargingStation': 500}

NC, NS = 2, 16
SB = 1024
NDC = NN['Customer']
NDC_PAD = 102400
ROWS_PT = NDC_PAD // NS
ND_SMALL = 1024

REL_INFO = [(n, s, d, d == 'Customer') for n, s, d in RELS]


def _ceil_to(x, m):
    return ((x + m - 1) // m) * m


def _pad_edges(ei, ne_pad, n_src, trash_base, trash_mod):
    ne = ei.shape[1]
    npad = ne_pad - ne
    i = jnp.arange(npad, dtype=jnp.int32)
    src = jnp.concatenate([ei[0], i % min(n_src, 4096)])
    dst = jnp.concatenate([ei[1], trash_base + (i % trash_mod)])
    srcs = tuple((src * 4 + g).reshape(-1, 128) for g in range(4))
    return srcs + (dst.reshape(-1, 128),)


def _build_sums_kernel(ne_pads):
    mesh = plsc.VectorSubcoreMesh(core_axis_name="c", subcore_axis_name="s",
                                  num_cores=NC, num_subcores=NS)
    out_type = [jax.ShapeDtypeStruct((NDC_PAD if big else ND_SMALL, HD),
                                     jnp.float32)
                for _, _, _, big in REL_INFO]
    scratch = [
        pltpu.VMEM_SHARED((NDC_PAD, 16), jnp.float32),
        pltpu.VMEM((8, 128), jnp.int32),
        pltpu.VMEM((8, 128), jnp.int32),
        pltpu.VMEM((SB, 16), jnp.float32),
        pltpu.SemaphoreType.DMA,
        pltpu.SemaphoreType.DMA,
    ]

    def body(*refs):
        it = iter(refs)
        xtabs4 = {nt: next(it) for nt in NODE_TYPES}
        edges = [tuple(next(it) for _ in range(5)) for _ in REL_INFO]
        zeros_c = next(it)
        S_out = [next(it) for _ in REL_INFO]
        acc, srcv, dstv, rows16, gsem, ssem = [next(it) for _ in range(6)]

        c = lax.axis_index("c")
        s = lax.axis_index("s")

        for ri, (name, st, dt, big) in enumerate(REL_INFO):
            dstb = edges[ri][4]
            xtab4 = xtabs4[st]
            nsb = ne_pads[name] // (NS * SB)
            rows_pt = (NDC_PAD if big else ND_SMALL) // NS
            for g in range(4):
                @pl.when(c == g // 2)
                def _(g=g, ri=ri, srcb=edges[ri][g], dstb=dstb, xtab4=xtab4,
                      nsb=nsb, rows_pt=rows_pt):
                    zcps = [pltpu.async_copy(
                        zeros_c.at[pl.ds(0, min(512, rows_pt - z * 512))],
                        acc.at[pl.ds(s * rows_pt + z * 512,
                                     min(512, rows_pt - z * 512))], gsem)
                        for z in range(-(-rows_pt // 512))]
                    for cp in zcps:
                        cp.wait()
                    plsc.subcore_barrier()

                    def sb_body(i, carry):
                        row0 = (s * nsb + i) * (SB // 128)
                        pltpu.sync_copy(srcb.at[pl.ds(row0, 8)], srcv)
                        pltpu.sync_copy(dstb.at[pl.ds(row0, 8)], dstv)
                        gcps = [pltpu.async_copy(
                            xtab4.at[srcv.at[j]],
                            rows16.at[pl.ds(j * 128, 128)], gsem)
                            for j in range(8)]
                        for cp in gcps:
                            cp.wait()
                        scps = [pltpu.async_copy(
                            rows16.at[pl.ds(j * 128, 128)],
                            acc.at[dstv.at[j]], ssem, add=True)
                            for j in range(8)]
                        for cp in scps:
                            cp.wait()
                        return carry

                    lax.fori_loop(0, nsb, sb_body, 0)
                    plsc.subcore_barrier()
                    pltpu.sync_copy(
                        acc.at[pl.ds(s * rows_pt, rows_pt)],
                        S_out[ri].at[pl.ds(s * rows_pt, rows_pt),
                                     pl.ds(16 * g, 16)])
                    plsc.subcore_barrier()

    return pl.kernel(body, out_type=out_type, mesh=mesh, scratch_types=scratch,
                     compiler_params=pltpu.CompilerParams(use_tc_tiling_on_sc=False))


def _build_counts_kernel(ne_pads):
    mesh = plsc.VectorSubcoreMesh(core_axis_name="c", subcore_axis_name="s",
                                  num_cores=NC, num_subcores=NS)
    out_type = [jax.ShapeDtypeStruct((NC, NDC_PAD if big else ND_SMALL),
                                     jnp.float32)
                for _, _, _, big in REL_INFO]
    scratch = [
        pltpu.VMEM_SHARED((NDC_PAD,), jnp.float32),
        pltpu.VMEM((8, 128), jnp.int32),
        pltpu.VMEM((128,), jnp.float32),
        pltpu.SemaphoreType.DMA,
        pltpu.SemaphoreType.DMA,
    ]

    def body(*refs):
        it = iter(refs)
        dstbs = [next(it) for _ in REL_INFO]
        zcnt = next(it)
        ones_h = next(it)
        C_out = [next(it) for _ in REL_INFO]
        cnt, dstv, onesv, gsem, csem = [next(it) for _ in range(5)]

        c = lax.axis_index("c")
        s = lax.axis_index("s")
        w = c * NS + s
        pltpu.sync_copy(ones_h, onesv)

        for ri, (name, st, dt, big) in enumerate(REL_INFO):
            nsb = ne_pads[name] // (NC * NS * SB)
            rows_pt = (NDC_PAD if big else ND_SMALL) // NS
            zcps = [pltpu.async_copy(
                zcnt.at[pl.ds(0, min(512, rows_pt - z * 512))],
                cnt.at[pl.ds(s * rows_pt + z * 512,
                             min(512, rows_pt - z * 512))], gsem)
                for z in range(-(-rows_pt // 512))]
            for cp in zcps:
                cp.wait()
            plsc.subcore_barrier()

            def sb_body(i, carry, dstb=dstbs[ri], nsb=nsb):
                row0 = (w * nsb + i) * (SB // 128)
                pltpu.sync_copy(dstb.at[pl.ds(row0, 8)], dstv)
                ccps = [pltpu.async_copy(
                    onesv, cnt.at[dstv.at[j]], csem, add=True)
                    for j in range(8)]
                for cp in ccps:
                    cp.wait()
                return carry

            lax.fori_loop(0, nsb, sb_body, 0)
            plsc.subcore_barrier()
            pltpu.sync_copy(cnt.at[pl.ds(s * rows_pt, rows_pt)],
                            C_out[ri].at[c, pl.ds(s * rows_pt, rows_pt)])
            plsc.subcore_barrier()

    return pl.kernel(body, out_type=out_type, mesh=mesh, scratch_types=scratch,
                     compiler_params=pltpu.CompilerParams(use_tc_tiling_on_sc=False))


def _seg_sums(x, edge_blocks):
    ne_pads = {name: blk[-1].shape[0] * 128 for name, blk in edge_blocks.items()}
    kern = _build_sums_kernel(ne_pads)
    args = [lax.optimization_barrier(x[nt].reshape(-1, 16)) for nt in NODE_TYPES]
    for name, _, _, _ in REL_INFO:
        args += list(edge_blocks[name])
    args.append(jnp.zeros((512, 16), jnp.float32))
    outs = kern(*args)
    return {name: outs[i] for i, (name, _, _, _) in enumerate(REL_INFO)}


def _seg_counts(edge_blocks):
    ne_pads = {name: blk[-1].shape[0] * 128 for name, blk in edge_blocks.items()}
    kern = _build_counts_kernel(ne_pads)
    args = [edge_blocks[name][4] for name, _, _, _ in REL_INFO]
    args += [jnp.zeros((512,), jnp.float32), jnp.ones((128,), jnp.float32)]
    outs = kern(*args)
    return {name: outs[i][0] + outs[i][1]
            for i, (name, _, _, _) in enumerate(REL_INFO)}


def _col(v):
    return v.reshape(1, -1)


def _proj(xf, W, b, br=None):
    n, k = xf.shape
    m = W.shape[1]

    def body(x_ref, w_ref, b_ref, o_ref):
        o_ref[...] = jax.nn.relu(
            jnp.dot(x_ref[...], w_ref[...], preferred_element_type=jnp.float32)
            + b_ref[...])

    if br is None:
        br, grid = n, 1
    else:
        grid = n // br
    return pl.pallas_call(
        body,
        grid=(grid,),
        in_specs=[pl.BlockSpec((br, k), lambda i: (i, 0)),
                  pl.BlockSpec((k, m), lambda i: (0, 0)),
                  pl.BlockSpec((1, m), lambda i: (0, 0))],
        out_specs=pl.BlockSpec((br, m), lambda i: (i, 0)),
        out_shape=jax.ShapeDtypeStruct((n, m), jnp.float32),
    )(xf, W, _col(b))


def _dense_layer(x, Ss, cnts, Wls, bls, Wrs, ln_g, ln_b, br=None):
    n = x.shape[0]
    nr = len(Ss)

    def body(*refs):
        x_ref = refs[0]
        S_refs = refs[1:1 + nr]
        c_refs = refs[1 + nr:1 + 2 * nr]
        wl_refs = refs[1 + 2 * nr:1 + 3 * nr]
        bl_refs = refs[1 + 3 * nr:1 + 4 * nr]
        wr_refs = refs[1 + 4 * nr:1 + 5 * nr]
        g_ref, b_ref, o_ref = refs[1 + 5 * nr:]
        x_blk = x_ref[...]
        acc = jnp.zeros_like(x_blk)
        for r in range(nr):
            inv = 1.0 / jnp.maximum(c_refs[r][...], 1.0)
            mean = S_refs[r][...] * inv
            o = (jnp.dot(mean, wl_refs[r][...], preferred_element_type=jnp.float32)
                 + bl_refs[r][...]
                 + jnp.dot(x_blk, wr_refs[r][...], preferred_element_type=jnp.float32))
            nrm = jnp.sqrt(jnp.sum(o * o, axis=-1, keepdims=True))
            acc = acc + o / jnp.maximum(nrm, 1e-12)
        y = acc + x_blk
        mu = jnp.mean(y, axis=-1, keepdims=True)
        var = jnp.mean((y - mu) ** 2, axis=-1, keepdims=True)
        o_ref[...] = (y - mu) / jnp.sqrt(var + 1e-5) * g_ref[...] + b_ref[...]

    if br is None:
        br, grid = n, 1
    else:
        grid = n // br
    row_spec = pl.BlockSpec((br, HD), lambda i: (i, 0))
    col_spec = pl.BlockSpec((br, 1), lambda i: (i, 0))
    full_spec = pl.BlockSpec((HD, HD), lambda i: (0, 0))
    vec_spec = pl.BlockSpec((1, HD), lambda i: (0, 0))
    return pl.pallas_call(
        body,
        grid=(grid,),
        in_specs=([row_spec] + [row_spec] * nr + [col_spec] * nr
                  + [full_spec] * nr + [vec_spec] * nr + [full_spec] * nr
                  + [vec_spec, vec_spec]),
        out_specs=row_spec,
        out_shape=jax.ShapeDtypeStruct((n, HD), jnp.float32),
    )(x, *Ss, *[c.reshape(-1, 1) for c in cnts], *Wls,
      *[_col(b) for b in bls], *Wrs, _col(ln_g), _col(ln_b))


def _attn_pool(cust, W, b, n_valid, br):
    n = cust.shape[0]
    grid = n // br

    def body(x_ref, w_ref, b_ref, o_ref, m_ref, d_ref, a_ref):
        i = pl.program_id(0)

        @pl.when(i == 0)
        def _():
            m_ref[...] = jnp.full((1, 1), -1e30, jnp.float32)
            d_ref[...] = jnp.zeros((1, 1), jnp.float32)
            a_ref[...] = jnp.zeros((1, HD), jnp.float32)

        x_blk = x_ref[...]
        logits = (jnp.dot(x_blk, w_ref[...], preferred_element_type=jnp.float32)
                  + b_ref[...])
        rows = i * br + lax.broadcasted_iota(jnp.int32, (br, 1), 0)
        logits = jnp.where(rows < n_valid, logits, -1e30)
        m_old = m_ref[...]
        m_new = jnp.maximum(m_old, jnp.max(logits))
        m_ref[...] = m_new
        scale = jnp.exp(m_old - m_new)
        e = jnp.where(rows < n_valid, jnp.exp(logits - m_new), 0.0)
        d_ref[...] = d_ref[...] * scale + jnp.sum(e)
        a_ref[...] = a_ref[...] * scale + lax.dot_general(
            e, x_blk, (((0,), (0,)), ((), ())),
            preferred_element_type=jnp.float32)

        @pl.when(i == grid - 1)
        def _():
            o_ref[...] = a_ref[...] / d_ref[...]

    return pl.pallas_call(
        body,
        grid=(grid,),
        in_specs=[pl.BlockSpec((br, HD), lambda i: (i, 0)),
                  pl.BlockSpec((HD, 1), lambda i: (0, 0)),
                  pl.BlockSpec((1, 1), lambda i: (0, 0))],
        out_specs=pl.BlockSpec((1, HD), lambda i: (0, 0)),
        out_shape=jax.ShapeDtypeStruct((1, HD), jnp.float32),
        scratch_shapes=[pltpu.VMEM((1, 1), jnp.float32),
                        pltpu.VMEM((1, 1), jnp.float32),
                        pltpu.VMEM((1, HD), jnp.float32)],
    )(cust, W, _col(b))


def _head(graph_vec, scalars, sp, op):
    def body(gv_ref, sc_ref, w1_ref, b1_ref, w2_ref, b2_ref,
             ow1_ref, ob1_ref, ow2_ref, ob2_ref, o_ref):
        sv = (jnp.dot(jax.nn.relu(
            jnp.dot(sc_ref[...], w1_ref[...], preferred_element_type=jnp.float32)
            + b1_ref[...]), w2_ref[...], preferred_element_type=jnp.float32)
            + b2_ref[...])
        h = jax.nn.relu(
            jnp.dot(gv_ref[...], ow1_ref[0:HD], preferred_element_type=jnp.float32)
            + jnp.dot(sv, ow1_ref[HD:2 * HD], preferred_element_type=jnp.float32)
            + ob1_ref[...])
        o_ref[...] = (jnp.dot(h, ow2_ref[...], preferred_element_type=jnp.float32)
                      + ob2_ref[...])

    out = pl.pallas_call(
        body,
        out_shape=jax.ShapeDtypeStruct((1, 128), jnp.float32),
    )(graph_vec, scalars, sp['W1'], _col(sp['b1']), sp['W2'], _col(sp['b2']),
      op['W1'], _col(op['b1']), op['W2'], _col(op['b2']))
    return out.reshape(128)


def kernel(x_depot, x_satellite, x_customer, x_rs, ei_cc_near, ei_cs_near, ei_sc_serves, ei_cc_follows, ei_dr_near, ei_sr_near, ei_cr_near, ei_rr_near, params, scalars):
    eis = {'cc_near': ei_cc_near, 'cs_near': ei_cs_near, 'sc_serves': ei_sc_serves, 'cc_follows': ei_cc_follows, 'dr_near': ei_dr_near, 'sr_near': ei_sr_near, 'cr_near': ei_cr_near, 'rr_near': ei_rr_near}
    xf = {'Depot': x_depot, 'Satellite': x_satellite, 'Customer': x_customer, 'RechargingStation': x_rs}

    edge_blocks = {}
    for name, st, dt, big in REL_INFO:
        ne_pad = _ceil_to(eis[name].shape[1], NC * NS * SB)
        trash_base = NDC if big else NN[dt]
        trash_mod = 2048 if big else 16
        edge_blocks[name] = _pad_edges(eis[name], ne_pad, NN[st], trash_base,
                                       trash_mod)

    pi = params['inp']
    x = {
        'Depot': _proj(x_depot, pi['Depot']['W'], pi['Depot']['b']),
        'Satellite': _proj(x_satellite, pi['Satellite']['W'], pi['Satellite']['b']),
        'Customer': _proj(jnp.pad(x_customer, ((0, NDC_PAD - NDC), (0, 0))),
                          pi['Customer']['W'], pi['Customer']['b'], br=1024),
        'RechargingStation': _proj(x_rs, pi['RechargingStation']['W'],
                                   pi['RechargingStation']['b']),
    }
    counts = _seg_counts(edge_blocks)

    by_dst = {'Customer': ['cc_near', 'sc_serves', 'cc_follows'],
              'Satellite': ['cs_near'],
              'RechargingStation': ['dr_near', 'sr_near', 'cr_near', 'rr_near']}
    for li in range(2):
        sums = _seg_sums(x, edge_blocks)
        cp = params['convs'][li]
        ln = params['ln'][li]
        newx = {}
        for dt, names in by_dst.items():
            nd = x[dt].shape[0]
            Ss = [sums[name][:nd] for name in names]
            cnts = [counts[name][:nd] for name in names]
            newx[dt] = _dense_layer(
                x[dt], Ss, cnts,
                [cp[n]['Wl'] for n in names], [cp[n]['bl'] for n in names],
                [cp[n]['Wr'] for n in names], ln[dt]['g'], ln[dt]['b'],
                br=1024 if dt == 'Customer' else None)
        x.update(newx)

    graph_vec = _attn_pool(x['Customer'], params['attn']['W'],
                           params['attn']['b'], NDC, 1024)
    return _head(graph_vec, scalars, params['scalar'], params['out'])


# speedup vs baseline: 1.0396x; 1.0396x over previous
import jax
import jax.numpy as jnp
from jax import lax
from jax.experimental import pallas as pl
from jax.experimental.pallas import tpu as pltpu
from jax.experimental.pallas import tpu_sc as plsc

HD = 64
NODE_TYPES = ['Depot', 'Satellite', 'Customer', 'RechargingStation']
RELS = [('cc_near', 'Customer', 'Customer'), ('cs_near', 'Customer', 'Satellite'), ('sc_serves', 'Satellite', 'Customer'), ('cc_follows', 'Customer', 'Customer'), ('dr_near', 'Depot', 'RechargingStation'), ('sr_near', 'Satellite', 'RechargingStation'), ('cr_near', 'Customer', 'RechargingStation'), ('rr_near', 'RechargingStation', 'RechargingStation')]
NN = {'Depot': 1, 'Satellite': 1000, 'Customer': 100000, 'RechargingStation': 500}

NC, NS = 2, 16
SB = 1024
NDC = NN['Customer']
NDC_PAD = 102400
ROWS_PT = NDC_PAD // NS
ND_SMALL = 1024

REL_INFO = [(n, s, d, d == 'Customer') for n, s, d in RELS]


def _ceil_to(x, m):
    return ((x + m - 1) // m) * m


def _pad_edges(ei, ne_pad, n_src, trash_base, trash_mod):
    ne = ei.shape[1]
    npad = ne_pad - ne
    i = jnp.arange(npad, dtype=jnp.int32)
    src = jnp.concatenate([ei[0], i % min(n_src, 4096)])
    dst = jnp.concatenate([ei[1], trash_base + (i % trash_mod)])
    srcs = tuple((src * 4 + g).reshape(-1, 128) for g in range(4))
    return srcs + (dst.reshape(-1, 128),)


def _build_sums_kernel(ne_pads, rel_info):
    mesh = plsc.VectorSubcoreMesh(core_axis_name="c", subcore_axis_name="s",
                                  num_cores=NC, num_subcores=NS)
    out_type = [jax.ShapeDtypeStruct((NDC_PAD if big else ND_SMALL, HD),
                                     jnp.float32)
                for _, _, _, big in rel_info]
    scratch = [
        pltpu.VMEM_SHARED((NDC_PAD, 16), jnp.float32),
        pltpu.VMEM((32, 128), jnp.int32),
        pltpu.VMEM((32, 128), jnp.int32),
        pltpu.VMEM((SB, 16), jnp.float32),
        pltpu.SemaphoreType.DMA,
        pltpu.SemaphoreType.DMA,
        pltpu.SemaphoreType.DMA,
    ]

    def body(*refs):
        it = iter(refs)
        xtabs4 = {nt: next(it) for nt in NODE_TYPES}
        edges = [tuple(next(it) for _ in range(5)) for _ in rel_info]
        zeros_c = next(it)
        S_out = [next(it) for _ in rel_info]
        acc, srcv, dstv, rows16, gsem, ssem, isem = [next(it) for _ in range(7)]

        c = lax.axis_index("c")
        s = lax.axis_index("s")

        for ri, (name, st, dt, big) in enumerate(rel_info):
            dstb = edges[ri][4]
            xtab4 = xtabs4[st]
            nsb = ne_pads[name] // (NS * SB)
            rows_pt = (NDC_PAD if big else ND_SMALL) // NS
            for g in range(4):
                @pl.when(c == g // 2)
                def _(g=g, ri=ri, srcb=edges[ri][g], dstb=dstb, xtab4=xtab4,
                      nsb=nsb, rows_pt=rows_pt):
                    if big:
                        def zero_body(z, carry):
                            pltpu.sync_copy(
                                zeros_c,
                                acc.at[pl.ds(s * rows_pt + z * 512, 512)])
                            return carry
                        lax.fori_loop(0, rows_pt // 512, zero_body, 0)
                        if rows_pt % 512:
                            pltpu.sync_copy(
                                zeros_c.at[pl.ds(0, rows_pt % 512)],
                                acc.at[pl.ds(s * rows_pt + rows_pt - rows_pt % 512,
                                             rows_pt % 512)])
                    else:
                        pltpu.sync_copy(zeros_c.at[pl.ds(0, rows_pt)],
                                        acc.at[pl.ds(s * rows_pt, rows_pt)])
                    plsc.subcore_barrier()

                    if big:
                        nsg = nsb // 2

                        def sb_body(i, carry):
                            row0 = (s * nsg + i) * 16
                            stg = [pltpu.async_copy(srcb.at[pl.ds(row0, 16)],
                                                    srcv.at[pl.ds(0, 16)], isem),
                                   pltpu.async_copy(dstb.at[pl.ds(row0, 16)],
                                                    dstv.at[pl.ds(0, 16)], isem)]
                            for cp in stg:
                                cp.wait()
                            pend = []
                            for hh in range(4):
                                buf = (hh % 2) * 512
                                if len(pend) == 2:
                                    for cp in pend.pop(0):
                                        cp.wait()
                                gcps = [pltpu.async_copy(
                                    xtab4.at[srcv.at[hh * 4 + j]],
                                    rows16.at[pl.ds(buf + j * 128, 128)], gsem)
                                    for j in range(4)]
                                for cp in gcps:
                                    cp.wait()
                                pend.append([pltpu.async_copy(
                                    rows16.at[pl.ds(buf + j * 128, 128)],
                                    acc.at[dstv.at[hh * 4 + j]], ssem, add=True)
                                    for j in range(4)])
                            for scps in pend:
                                for cp in scps:
                                    cp.wait()
                            return carry
                    else:
                        nsg = nsb

                        def sb_body(i, carry):
                            row0 = (s * nsg + i) * 8
                            pltpu.sync_copy(srcb.at[pl.ds(row0, 8)],
                                            srcv.at[pl.ds(0, 8)])
                            pltpu.sync_copy(dstb.at[pl.ds(row0, 8)],
                                            dstv.at[pl.ds(0, 8)])
                            gcps = [pltpu.async_copy(
                                xtab4.at[srcv.at[j]],
                                rows16.at[pl.ds(j * 128, 128)], gsem)
                                for j in range(8)]
                            for cp in gcps:
                                cp.wait()
                            scps = [pltpu.async_copy(
                                rows16.at[pl.ds(j * 128, 128)],
                                acc.at[dstv.at[j]], ssem, add=True)
                                for j in range(8)]
                            for cp in scps:
                                cp.wait()
                            return carry

                    lax.fori_loop(0, nsg, sb_body, 0)
                    plsc.subcore_barrier()
                    pltpu.sync_copy(
                        acc.at[pl.ds(s * rows_pt, rows_pt)],
                        S_out[ri].at[pl.ds(s * rows_pt, rows_pt),
                                     pl.ds(16 * g, 16)])
                    plsc.subcore_barrier()

    return pl.kernel(body, out_type=out_type, mesh=mesh, scratch_types=scratch,
                     compiler_params=pltpu.CompilerParams(use_tc_tiling_on_sc=False))


def _build_counts_kernel(ne_pads):
    mesh = plsc.VectorSubcoreMesh(core_axis_name="c", subcore_axis_name="s",
                                  num_cores=NC, num_subcores=NS)
    out_type = [jax.ShapeDtypeStruct((NC, NDC_PAD if big else ND_SMALL),
                                     jnp.float32)
                for _, _, _, big in REL_INFO]
    scratch = [
        pltpu.VMEM_SHARED((NDC_PAD,), jnp.float32),
        pltpu.VMEM((8, 128), jnp.int32),
        pltpu.VMEM((128,), jnp.float32),
        pltpu.SemaphoreType.DMA,
        pltpu.SemaphoreType.DMA,
    ]

    def body(*refs):
        it = iter(refs)
        dstbs = [next(it) for _ in REL_INFO]
        zcnt = next(it)
        ones_h = next(it)
        C_out = [next(it) for _ in REL_INFO]
        cnt, dstv, onesv, gsem, csem = [next(it) for _ in range(5)]

        c = lax.axis_index("c")
        s = lax.axis_index("s")
        w = c * NS + s
        pltpu.sync_copy(ones_h, onesv)

        for ri, (name, st, dt, big) in enumerate(REL_INFO):
            nsb = ne_pads[name] // (NC * NS * SB)
            rows_pt = (NDC_PAD if big else ND_SMALL) // NS
            zcps = [pltpu.async_copy(
                zcnt.at[pl.ds(0, min(512, rows_pt - z * 512))],
                cnt.at[pl.ds(s * rows_pt + z * 512,
                             min(512, rows_pt - z * 512))], gsem)
                for z in range(-(-rows_pt // 512))]
            for cp in zcps:
                cp.wait()
            plsc.subcore_barrier()

            def sb_body(i, carry, dstb=dstbs[ri], nsb=nsb):
                row0 = (w * nsb + i) * (SB // 128)
                pltpu.sync_copy(dstb.at[pl.ds(row0, 8)], dstv)
                ccps = [pltpu.async_copy(
                    onesv, cnt.at[dstv.at[j]], csem, add=True)
                    for j in range(8)]
                for cp in ccps:
                    cp.wait()
                return carry

            lax.fori_loop(0, nsb, sb_body, 0)
            plsc.subcore_barrier()
            pltpu.sync_copy(cnt.at[pl.ds(s * rows_pt, rows_pt)],
                            C_out[ri].at[c, pl.ds(s * rows_pt, rows_pt)])
            plsc.subcore_barrier()

    return pl.kernel(body, out_type=out_type, mesh=mesh, scratch_types=scratch,
                     compiler_params=pltpu.CompilerParams(use_tc_tiling_on_sc=False))


def _seg_sums(x, edge_blocks):
    ne_pads = {name: blk[-1].shape[0] * 128 for name, blk in edge_blocks.items()}
    xtabs = [lax.optimization_barrier(x[nt].reshape(-1, 16)) for nt in NODE_TYPES]
    sums = {}
    for rel_info in (REL_INFO[:2], REL_INFO[2:]):
        kern = _build_sums_kernel(ne_pads, rel_info)
        args = list(xtabs)
        for name, _, _, _ in rel_info:
            args += list(edge_blocks[name])
        args.append(jnp.zeros((512, 16), jnp.float32))
        outs = kern(*args)
        sums.update({name: outs[i] for i, (name, _, _, _) in enumerate(rel_info)})
    return sums


def _seg_counts(edge_blocks):
    ne_pads = {name: blk[-1].shape[0] * 128 for name, blk in edge_blocks.items()}
    kern = _build_counts_kernel(ne_pads)
    args = [edge_blocks[name][4] for name, _, _, _ in REL_INFO]
    args += [jnp.zeros((512,), jnp.float32), jnp.ones((128,), jnp.float32)]
    outs = kern(*args)
    return {name: outs[i][0] + outs[i][1]
            for i, (name, _, _, _) in enumerate(REL_INFO)}


def _col(v):
    return v.reshape(1, -1)


def _proj(xf, W, b, br=None):
    n, k = xf.shape
    m = W.shape[1]

    def body(x_ref, w_ref, b_ref, o_ref):
        o_ref[...] = jax.nn.relu(
            jnp.dot(x_ref[...], w_ref[...], preferred_element_type=jnp.float32)
            + b_ref[...])

    if br is None:
        br, grid = n, 1
    else:
        grid = n // br
    return pl.pallas_call(
        body,
        grid=(grid,),
        in_specs=[pl.BlockSpec((br, k), lambda i: (i, 0)),
                  pl.BlockSpec((k, m), lambda i: (0, 0)),
                  pl.BlockSpec((1, m), lambda i: (0, 0))],
        out_specs=pl.BlockSpec((br, m), lambda i: (i, 0)),
        out_shape=jax.ShapeDtypeStruct((n, m), jnp.float32),
    )(xf, W, _col(b))


def _dense_layer(x, Ss, cnts, Wls, bls, Wrs, ln_g, ln_b, br=None):
    n = x.shape[0]
    nr = len(Ss)

    def body(*refs):
        x_ref = refs[0]
        S_refs = refs[1:1 + nr]
        c_refs = refs[1 + nr:1 + 2 * nr]
        wl_refs = refs[1 + 2 * nr:1 + 3 * nr]
        bl_refs = refs[1 + 3 * nr:1 + 4 * nr]
        wr_refs = refs[1 + 4 * nr:1 + 5 * nr]
        g_ref, b_ref, o_ref = refs[1 + 5 * nr:]
        x_blk = x_ref[...]
        acc = jnp.zeros_like(x_blk)
        for r in range(nr):
            inv = 1.0 / jnp.maximum(c_refs[r][...], 1.0)
            mean = S_refs[r][...] * inv
            o = (jnp.dot(mean, wl_refs[r][...], preferred_element_type=jnp.float32)
                 + bl_refs[r][...]
                 + jnp.dot(x_blk, wr_refs[r][...], preferred_element_type=jnp.float32))
            nrm = jnp.sqrt(jnp.sum(o * o, axis=-1, keepdims=True))
            acc = acc + o / jnp.maximum(nrm, 1e-12)
        y = acc + x_blk
        mu = jnp.mean(y, axis=-1, keepdims=True)
        var = jnp.mean((y - mu) ** 2, axis=-1, keepdims=True)
        o_ref[...] = (y - mu) / jnp.sqrt(var + 1e-5) * g_ref[...] + b_ref[...]

    if br is None:
        br, grid = n, 1
    else:
        grid = n // br
    row_spec = pl.BlockSpec((br, HD), lambda i: (i, 0))
    col_spec = pl.BlockSpec((br, 1), lambda i: (i, 0))
    full_spec = pl.BlockSpec((HD, HD), lambda i: (0, 0))
    vec_spec = pl.BlockSpec((1, HD), lambda i: (0, 0))
    return pl.pallas_call(
        body,
        grid=(grid,),
        in_specs=([row_spec] + [row_spec] * nr + [col_spec] * nr
                  + [full_spec] * nr + [vec_spec] * nr + [full_spec] * nr
                  + [vec_spec, vec_spec]),
        out_specs=row_spec,
        out_shape=jax.ShapeDtypeStruct((n, HD), jnp.float32),
    )(x, *Ss, *[c.reshape(-1, 1) for c in cnts], *Wls,
      *[_col(b) for b in bls], *Wrs, _col(ln_g), _col(ln_b))


def _attn_pool(cust, W, b, n_valid, br):
    n = cust.shape[0]
    grid = n // br

    def body(x_ref, w_ref, b_ref, o_ref, m_ref, d_ref, a_ref):
        i = pl.program_id(0)

        @pl.when(i == 0)
        def _():
            m_ref[...] = jnp.full((1, 1), -1e30, jnp.float32)
            d_ref[...] = jnp.zeros((1, 1), jnp.float32)
            a_ref[...] = jnp.zeros((1, HD), jnp.float32)

        x_blk = x_ref[...]
        logits = (jnp.dot(x_blk, w_ref[...], preferred_element_type=jnp.float32)
                  + b_ref[...])
        rows = i * br + lax.broadcasted_iota(jnp.int32, (br, 1), 0)
        logits = jnp.where(rows < n_valid, logits, -1e30)
        m_old = m_ref[...]
        m_new = jnp.maximum(m_old, jnp.max(logits))
        m_ref[...] = m_new
        scale = jnp.exp(m_old - m_new)
        e = jnp.where(rows < n_valid, jnp.exp(logits - m_new), 0.0)
        d_ref[...] = d_ref[...] * scale + jnp.sum(e)
        a_ref[...] = a_ref[...] * scale + lax.dot_general(
            e, x_blk, (((0,), (0,)), ((), ())),
            preferred_element_type=jnp.float32)

        @pl.when(i == grid - 1)
        def _():
            o_ref[...] = a_ref[...] / d_ref[...]

    return pl.pallas_call(
        body,
        grid=(grid,),
        in_specs=[pl.BlockSpec((br, HD), lambda i: (i, 0)),
                  pl.BlockSpec((HD, 1), lambda i: (0, 0)),
                  pl.BlockSpec((1, 1), lambda i: (0, 0))],
        out_specs=pl.BlockSpec((1, HD), lambda i: (0, 0)),
        out_shape=jax.ShapeDtypeStruct((1, HD), jnp.float32),
        scratch_shapes=[pltpu.VMEM((1, 1), jnp.float32),
                        pltpu.VMEM((1, 1), jnp.float32),
                        pltpu.VMEM((1, HD), jnp.float32)],
    )(cust, W, _col(b))


def _head(graph_vec, scalars, sp, op):
    def body(gv_ref, sc_ref, w1_ref, b1_ref, w2_ref, b2_ref,
             ow1_ref, ob1_ref, ow2_ref, ob2_ref, o_ref):
        sv = (jnp.dot(jax.nn.relu(
            jnp.dot(sc_ref[...], w1_ref[...], preferred_element_type=jnp.float32)
            + b1_ref[...]), w2_ref[...], preferred_element_type=jnp.float32)
            + b2_ref[...])
        h = jax.nn.relu(
            jnp.dot(gv_ref[...], ow1_ref[0:HD], preferred_element_type=jnp.float32)
            + jnp.dot(sv, ow1_ref[HD:2 * HD], preferred_element_type=jnp.float32)
            + ob1_ref[...])
        o_ref[...] = (jnp.dot(h, ow2_ref[...], preferred_element_type=jnp.float32)
                      + ob2_ref[...])

    out = pl.pallas_call(
        body,
        out_shape=jax.ShapeDtypeStruct((1, 128), jnp.float32),
    )(graph_vec, scalars, sp['W1'], _col(sp['b1']), sp['W2'], _col(sp['b2']),
      op['W1'], _col(op['b1']), op['W2'], _col(op['b2']))
    return out.reshape(128)


def kernel(x_depot, x_satellite, x_customer, x_rs, ei_cc_near, ei_cs_near, ei_sc_serves, ei_cc_follows, ei_dr_near, ei_sr_near, ei_cr_near, ei_rr_near, params, scalars):
    eis = {'cc_near': ei_cc_near, 'cs_near': ei_cs_near, 'sc_serves': ei_sc_serves, 'cc_follows': ei_cc_follows, 'dr_near': ei_dr_near, 'sr_near': ei_sr_near, 'cr_near': ei_cr_near, 'rr_near': ei_rr_near}
    xf = {'Depot': x_depot, 'Satellite': x_satellite, 'Customer': x_customer, 'RechargingStation': x_rs}

    edge_blocks = {}
    for name, st, dt, big in REL_INFO:
        ne_pad = _ceil_to(eis[name].shape[1], 2 * NS * SB)
        trash_base = NDC if big else NN[dt]
        trash_mod = 2048 if big else 16
        edge_blocks[name] = _pad_edges(eis[name], ne_pad, NN[st], trash_base,
                                       trash_mod)

    pi = params['inp']
    x = {
        'Depot': _proj(x_depot, pi['Depot']['W'], pi['Depot']['b']),
        'Satellite': _proj(x_satellite, pi['Satellite']['W'], pi['Satellite']['b']),
        'Customer': _proj(jnp.pad(x_customer, ((0, NDC_PAD - NDC), (0, 0))),
                          pi['Customer']['W'], pi['Customer']['b'], br=1024),
        'RechargingStation': _proj(x_rs, pi['RechargingStation']['W'],
                                   pi['RechargingStation']['b']),
    }
    counts = _seg_counts(edge_blocks)

    by_dst = {'Customer': ['cc_near', 'sc_serves', 'cc_follows'],
              'Satellite': ['cs_near'],
              'RechargingStation': ['dr_near', 'sr_near', 'cr_near', 'rr_near']}
    for li in range(2):
        sums = _seg_sums(x, edge_blocks)
        cp = params['convs'][li]
        ln = params['ln'][li]
        newx = {}
        for dt, names in by_dst.items():
            nd = x[dt].shape[0]
            Ss = [sums[name][:nd] for name in names]
            cnts = [counts[name][:nd] for name in names]
            newx[dt] = _dense_layer(
                x[dt], Ss, cnts,
                [cp[n]['Wl'] for n in names], [cp[n]['bl'] for n in names],
                [cp[n]['Wr'] for n in names], ln[dt]['g'], ln[dt]['b'],
                br=1024 if dt == 'Customer' else None)
        x.update(newx)

    graph_vec = _attn_pool(x['Customer'], params['attn']['W'],
                           params['attn']['b'], NDC, 1024)
    return _head(graph_vec, scalars, params['scalar'], params['out'])


# speedup vs baseline: 1.4003x; 1.3469x over previous
import jax
import jax.numpy as jnp
from jax import lax
from jax.experimental import pallas as pl
from jax.experimental.pallas import tpu as pltpu
from jax.experimental.pallas import tpu_sc as plsc

HD = 64
NODE_TYPES = ['Depot', 'Satellite', 'Customer', 'RechargingStation']
RELS = [('cc_near', 'Customer', 'Customer'), ('cs_near', 'Customer', 'Satellite'), ('sc_serves', 'Satellite', 'Customer'), ('cc_follows', 'Customer', 'Customer'), ('dr_near', 'Depot', 'RechargingStation'), ('sr_near', 'Satellite', 'RechargingStation'), ('cr_near', 'Customer', 'RechargingStation'), ('rr_near', 'RechargingStation', 'RechargingStation')]
NN = {'Depot': 1, 'Satellite': 1000, 'Customer': 100000, 'RechargingStation': 500}

NC, NS = 2, 16
SB = 1024
NDC = NN['Customer']
NDC_PAD = 102400
ROWS_PT = NDC_PAD // NS
ND_SMALL = 1024

REL_INFO = [(n, s, d, d == 'Customer') for n, s, d in RELS]


def _ceil_to(x, m):
    return ((x + m - 1) // m) * m


def _pad_edges(ei, ne_pad, n_src, trash_base, trash_mod):
    ne = ei.shape[1]
    npad = ne_pad - ne
    i = jnp.arange(npad, dtype=jnp.int32)
    src = jnp.concatenate([ei[0], i % min(n_src, 4096)])
    dst = jnp.concatenate([ei[1], trash_base + (i % trash_mod)])
    srcs = tuple((src * 4 + g).reshape(-1, 128) for g in range(4))
    return srcs + (dst.reshape(-1, 128),)


def _build_sums_kernel(ne_pads, rel_info):
    mesh = plsc.VectorSubcoreMesh(core_axis_name="c", subcore_axis_name="s",
                                  num_cores=NC, num_subcores=NS)
    out_type = [jax.ShapeDtypeStruct((NDC_PAD if big else ND_SMALL, HD),
                                     jnp.float32)
                for _, _, _, big in rel_info]
    scratch = [
        pltpu.VMEM_SHARED((NDC_PAD, 16), jnp.float32),
        pltpu.VMEM((32, 128), jnp.int32),
        pltpu.VMEM((32, 128), jnp.int32),
        pltpu.VMEM((SB, 16), jnp.float32),
        pltpu.SemaphoreType.DMA,
        pltpu.SemaphoreType.DMA,
        pltpu.SemaphoreType.DMA,
    ]

    def body(*refs):
        it = iter(refs)
        xtabs4 = {nt: next(it) for nt in NODE_TYPES}
        edges = [tuple(next(it) for _ in range(5)) for _ in rel_info]
        zeros_c = next(it)
        S_out = [next(it) for _ in rel_info]
        acc, srcv, dstv, rows16, gsem, ssem, isem = [next(it) for _ in range(7)]

        c = lax.axis_index("c")
        s = lax.axis_index("s")

        for ri, (name, st, dt, big) in enumerate(rel_info):
            dstb = edges[ri][4]
            xtab4 = xtabs4[st]
            nsb = ne_pads[name] // (NS * SB)
            rows_pt = (NDC_PAD if big else ND_SMALL) // NS
            for g in range(4):
                @pl.when(c == g // 2)
                def _(g=g, ri=ri, srcb=edges[ri][g], dstb=dstb, xtab4=xtab4,
                      nsb=nsb, rows_pt=rows_pt):
                    if big:
                        def zero_body(z, carry):
                            pltpu.sync_copy(
                                zeros_c,
                                acc.at[pl.ds(s * rows_pt + z * 512, 512)])
                            return carry
                        lax.fori_loop(0, rows_pt // 512, zero_body, 0)
                        if rows_pt % 512:
                            pltpu.sync_copy(
                                zeros_c.at[pl.ds(0, rows_pt % 512)],
                                acc.at[pl.ds(s * rows_pt + rows_pt - rows_pt % 512,
                                             rows_pt % 512)])
                    else:
                        pltpu.sync_copy(zeros_c.at[pl.ds(0, rows_pt)],
                                        acc.at[pl.ds(s * rows_pt, rows_pt)])
                    plsc.subcore_barrier()

                    if big:
                        nsg = nsb // 2

                        def sb_body(i, carry):
                            row0 = (s * nsg + i) * 16
                            stg = [pltpu.async_copy(srcb.at[pl.ds(row0, 16)],
                                                    srcv.at[pl.ds(0, 16)], isem),
                                   pltpu.async_copy(dstb.at[pl.ds(row0, 16)],
                                                    dstv.at[pl.ds(0, 16)], isem)]
                            for cp in stg:
                                cp.wait()
                            pend = []
                            for hh in range(4):
                                buf = (hh % 2) * 512
                                if len(pend) == 2:
                                    for cp in pend.pop(0):
                                        cp.wait()
                                gcps = [pltpu.async_copy(
                                    xtab4.at[srcv.at[hh * 4 + j]],
                                    rows16.at[pl.ds(buf + j * 128, 128)], gsem)
                                    for j in range(4)]
                                for cp in gcps:
                                    cp.wait()
                                pend.append([pltpu.async_copy(
                                    rows16.at[pl.ds(buf + j * 128, 128)],
                                    acc.at[dstv.at[hh * 4 + j]], ssem, add=True)
                                    for j in range(4)])
                            for scps in pend:
                                for cp in scps:
                                    cp.wait()
                            return carry
                    else:
                        nsg = nsb

                        def sb_body(i, carry):
                            row0 = (s * nsg + i) * 8
                            pltpu.sync_copy(srcb.at[pl.ds(row0, 8)],
                                            srcv.at[pl.ds(0, 8)])
                            pltpu.sync_copy(dstb.at[pl.ds(row0, 8)],
                                            dstv.at[pl.ds(0, 8)])
                            gcps = [pltpu.async_copy(
                                xtab4.at[srcv.at[j]],
                                rows16.at[pl.ds(j * 128, 128)], gsem)
                                for j in range(8)]
                            for cp in gcps:
                                cp.wait()
                            scps = [pltpu.async_copy(
                                rows16.at[pl.ds(j * 128, 128)],
                                acc.at[dstv.at[j]], ssem, add=True)
                                for j in range(8)]
                            for cp in scps:
                                cp.wait()
                            return carry

                    lax.fori_loop(0, nsg, sb_body, 0)
                    plsc.subcore_barrier()
                    pltpu.sync_copy(
                        acc.at[pl.ds(s * rows_pt, rows_pt)],
                        S_out[ri].at[pl.ds(s * rows_pt, rows_pt),
                                     pl.ds(16 * g, 16)])
                    plsc.subcore_barrier()

    return pl.kernel(body, out_type=out_type, mesh=mesh, scratch_types=scratch,
                     compiler_params=pltpu.CompilerParams(use_tc_tiling_on_sc=False))


def _build_counts_kernel(ne_pads):
    mesh = plsc.VectorSubcoreMesh(core_axis_name="c", subcore_axis_name="s",
                                  num_cores=NC, num_subcores=NS)
    out_type = [jax.ShapeDtypeStruct((NC, NDC_PAD if big else ND_SMALL),
                                     jnp.float32)
                for _, _, _, big in REL_INFO]
    scratch = [
        pltpu.VMEM_SHARED((NDC_PAD,), jnp.float32),
        pltpu.VMEM((8, 128), jnp.int32),
        pltpu.VMEM((128,), jnp.float32),
        pltpu.SemaphoreType.DMA,
        pltpu.SemaphoreType.DMA,
    ]

    def body(*refs):
        it = iter(refs)
        dstbs = [next(it) for _ in REL_INFO]
        zcnt = next(it)
        ones_h = next(it)
        C_out = [next(it) for _ in REL_INFO]
        cnt, dstv, onesv, gsem, csem = [next(it) for _ in range(5)]

        c = lax.axis_index("c")
        s = lax.axis_index("s")
        w = c * NS + s
        pltpu.sync_copy(ones_h, onesv)

        for ri, (name, st, dt, big) in enumerate(REL_INFO):
            nsb = ne_pads[name] // (NC * NS * 512)
            rows_pt = (NDC_PAD if big else ND_SMALL) // NS
            zcps = [pltpu.async_copy(
                zcnt.at[pl.ds(0, min(512, rows_pt - z * 512))],
                cnt.at[pl.ds(s * rows_pt + z * 512,
                             min(512, rows_pt - z * 512))], gsem)
                for z in range(-(-rows_pt // 512))]
            for cp in zcps:
                cp.wait()
            plsc.subcore_barrier()

            def sb_body(i, carry, dstb=dstbs[ri], nsb=nsb):
                row0 = (w * nsb + i) * 4
                pltpu.sync_copy(dstb.at[pl.ds(row0, 4)], dstv.at[pl.ds(0, 4)])
                ccps = [pltpu.async_copy(
                    onesv, cnt.at[dstv.at[j]], csem, add=True)
                    for j in range(4)]
                for cp in ccps:
                    cp.wait()
                return carry

            lax.fori_loop(0, nsb, sb_body, 0)
            plsc.subcore_barrier()
            pltpu.sync_copy(cnt.at[pl.ds(s * rows_pt, rows_pt)],
                            C_out[ri].at[c, pl.ds(s * rows_pt, rows_pt)])
            plsc.subcore_barrier()

    return pl.kernel(body, out_type=out_type, mesh=mesh, scratch_types=scratch,
                     compiler_params=pltpu.CompilerParams(use_tc_tiling_on_sc=False))


def _seg_sums(x, edge_blocks):
    ne_pads = {name: blk[-1].shape[0] * 128 for name, blk in edge_blocks.items()}
    xtabs = [lax.optimization_barrier(x[nt].reshape(-1, 16)) for nt in NODE_TYPES]
    sum_rels = [r for r in REL_INFO if r[0] != 'dr_near']
    sums = {}
    for rel_info in (sum_rels[:2], sum_rels[2:]):
        kern = _build_sums_kernel(ne_pads, rel_info)
        args = list(xtabs)
        for name, _, _, _ in rel_info:
            args += list(edge_blocks[name])
        args.append(jnp.zeros((512, 16), jnp.float32))
        outs = kern(*args)
        sums.update({name: outs[i] for i, (name, _, _, _) in enumerate(rel_info)})
    return sums


def _seg_counts(edge_blocks):
    ne_pads = {name: blk[-1].shape[0] * 128 for name, blk in edge_blocks.items()}
    kern = _build_counts_kernel(ne_pads)
    args = [edge_blocks[name][4] for name, _, _, _ in REL_INFO]
    args += [jnp.zeros((512,), jnp.float32), jnp.ones((128,), jnp.float32)]
    outs = kern(*args)
    return {name: outs[i][0] + outs[i][1]
            for i, (name, _, _, _) in enumerate(REL_INFO)}


def _col(v):
    return v.reshape(1, -1)


def _proj(xf, W, b, br=None):
    n, k = xf.shape
    m = W.shape[1]

    def body(x_ref, w_ref, b_ref, o_ref):
        o_ref[...] = jax.nn.relu(
            jnp.dot(x_ref[...], w_ref[...], preferred_element_type=jnp.float32)
            + b_ref[...])

    if br is None:
        br, grid = n, 1
    else:
        grid = n // br
    return pl.pallas_call(
        body,
        grid=(grid,),
        in_specs=[pl.BlockSpec((br, k), lambda i: (i, 0)),
                  pl.BlockSpec((k, m), lambda i: (0, 0)),
                  pl.BlockSpec((1, m), lambda i: (0, 0))],
        out_specs=pl.BlockSpec((br, m), lambda i: (i, 0)),
        out_shape=jax.ShapeDtypeStruct((n, m), jnp.float32),
    )(xf, W, _col(b))


def _dense_layer(x, Ss, cnts, Wls, bls, Wrs, ln_g, ln_b, br=None):
    n = x.shape[0]
    nr = len(Ss)

    def body(*refs):
        x_ref = refs[0]
        S_refs = refs[1:1 + nr]
        c_refs = refs[1 + nr:1 + 2 * nr]
        wl_refs = refs[1 + 2 * nr:1 + 3 * nr]
        bl_refs = refs[1 + 3 * nr:1 + 4 * nr]
        wr_refs = refs[1 + 4 * nr:1 + 5 * nr]
        g_ref, b_ref, o_ref = refs[1 + 5 * nr:]
        x_blk = x_ref[...]
        acc = jnp.zeros_like(x_blk)
        for r in range(nr):
            inv = 1.0 / jnp.maximum(c_refs[r][...], 1.0)
            mean = S_refs[r][...] * inv
            o = (jnp.dot(mean, wl_refs[r][...], preferred_element_type=jnp.float32)
                 + bl_refs[r][...]
                 + jnp.dot(x_blk, wr_refs[r][...], preferred_element_type=jnp.float32))
            nrm = jnp.sqrt(jnp.sum(o * o, axis=-1, keepdims=True))
            acc = acc + o / jnp.maximum(nrm, 1e-12)
        y = acc + x_blk
        mu = jnp.mean(y, axis=-1, keepdims=True)
        var = jnp.mean((y - mu) ** 2, axis=-1, keepdims=True)
        o_ref[...] = (y - mu) / jnp.sqrt(var + 1e-5) * g_ref[...] + b_ref[...]

    if br is None:
        br, grid = n, 1
    else:
        grid = n // br
    row_spec = pl.BlockSpec((br, HD), lambda i: (i, 0))
    col_spec = pl.BlockSpec((br, 1), lambda i: (i, 0))
    full_spec = pl.BlockSpec((HD, HD), lambda i: (0, 0))
    vec_spec = pl.BlockSpec((1, HD), lambda i: (0, 0))
    return pl.pallas_call(
        body,
        grid=(grid,),
        in_specs=([row_spec] + [row_spec] * nr + [col_spec] * nr
                  + [full_spec] * nr + [vec_spec] * nr + [full_spec] * nr
                  + [vec_spec, vec_spec]),
        out_specs=row_spec,
        out_shape=jax.ShapeDtypeStruct((n, HD), jnp.float32),
    )(x, *Ss, *[c.reshape(-1, 1) for c in cnts], *Wls,
      *[_col(b) for b in bls], *Wrs, _col(ln_g), _col(ln_b))


def _attn_pool(cust, W, b, n_valid, br):
    n = cust.shape[0]
    grid = n // br

    def body(x_ref, w_ref, b_ref, o_ref, m_ref, d_ref, a_ref):
        i = pl.program_id(0)

        @pl.when(i == 0)
        def _():
            m_ref[...] = jnp.full((1, 1), -1e30, jnp.float32)
            d_ref[...] = jnp.zeros((1, 1), jnp.float32)
            a_ref[...] = jnp.zeros((1, HD), jnp.float32)

        x_blk = x_ref[...]
        logits = (jnp.dot(x_blk, w_ref[...], preferred_element_type=jnp.float32)
                  + b_ref[...])
        rows = i * br + lax.broadcasted_iota(jnp.int32, (br, 1), 0)
        logits = jnp.where(rows < n_valid, logits, -1e30)
        m_old = m_ref[...]
        m_new = jnp.maximum(m_old, jnp.max(logits))
        m_ref[...] = m_new
        scale = jnp.exp(m_old - m_new)
        e = jnp.where(rows < n_valid, jnp.exp(logits - m_new), 0.0)
        d_ref[...] = d_ref[...] * scale + jnp.sum(e)
        a_ref[...] = a_ref[...] * scale + lax.dot_general(
            e, x_blk, (((0,), (0,)), ((), ())),
            preferred_element_type=jnp.float32)

        @pl.when(i == grid - 1)
        def _():
            o_ref[...] = a_ref[...] / d_ref[...]

    return pl.pallas_call(
        body,
        grid=(grid,),
        in_specs=[pl.BlockSpec((br, HD), lambda i: (i, 0)),
                  pl.BlockSpec((HD, 1), lambda i: (0, 0)),
                  pl.BlockSpec((1, 1), lambda i: (0, 0))],
        out_specs=pl.BlockSpec((1, HD), lambda i: (0, 0)),
        out_shape=jax.ShapeDtypeStruct((1, HD), jnp.float32),
        scratch_shapes=[pltpu.VMEM((1, 1), jnp.float32),
                        pltpu.VMEM((1, 1), jnp.float32),
                        pltpu.VMEM((1, HD), jnp.float32)],
    )(cust, W, _col(b))


def _head(graph_vec, scalars, sp, op):
    def body(gv_ref, sc_ref, w1_ref, b1_ref, w2_ref, b2_ref,
             ow1_ref, ob1_ref, ow2_ref, ob2_ref, o_ref):
        sv = (jnp.dot(jax.nn.relu(
            jnp.dot(sc_ref[...], w1_ref[...], preferred_element_type=jnp.float32)
            + b1_ref[...]), w2_ref[...], preferred_element_type=jnp.float32)
            + b2_ref[...])
        h = jax.nn.relu(
            jnp.dot(gv_ref[...], ow1_ref[0:HD], preferred_element_type=jnp.float32)
            + jnp.dot(sv, ow1_ref[HD:2 * HD], preferred_element_type=jnp.float32)
            + ob1_ref[...])
        o_ref[...] = (jnp.dot(h, ow2_ref[...], preferred_element_type=jnp.float32)
                      + ob2_ref[...])

    out = pl.pallas_call(
        body,
        out_shape=jax.ShapeDtypeStruct((1, 128), jnp.float32),
    )(graph_vec, scalars, sp['W1'], _col(sp['b1']), sp['W2'], _col(sp['b2']),
      op['W1'], _col(op['b1']), op['W2'], _col(op['b2']))
    return out.reshape(128)


def kernel(x_depot, x_satellite, x_customer, x_rs, ei_cc_near, ei_cs_near, ei_sc_serves, ei_cc_follows, ei_dr_near, ei_sr_near, ei_cr_near, ei_rr_near, params, scalars):
    eis = {'cc_near': ei_cc_near, 'cs_near': ei_cs_near, 'sc_serves': ei_sc_serves, 'cc_follows': ei_cc_follows, 'dr_near': ei_dr_near, 'sr_near': ei_sr_near, 'cr_near': ei_cr_near, 'rr_near': ei_rr_near}
    xf = {'Depot': x_depot, 'Satellite': x_satellite, 'Customer': x_customer, 'RechargingStation': x_rs}

    edge_blocks = {}
    for name, st, dt, big in REL_INFO:
        ne_pad = _ceil_to(eis[name].shape[1], (2 * NS * SB) if big else (NS * SB))
        trash_base = NDC if big else NN[dt]
        trash_mod = 2048 if big else 16
        edge_blocks[name] = _pad_edges(eis[name], ne_pad, NN[st], trash_base,
                                       trash_mod)

    pi = params['inp']
    x = {
        'Depot': _proj(x_depot, pi['Depot']['W'], pi['Depot']['b']),
        'Satellite': _proj(x_satellite, pi['Satellite']['W'], pi['Satellite']['b']),
        'Customer': _proj(jnp.pad(x_customer, ((0, NDC_PAD - NDC), (0, 0))),
                          pi['Customer']['W'], pi['Customer']['b'], br=1024),
        'RechargingStation': _proj(x_rs, pi['RechargingStation']['W'],
                                   pi['RechargingStation']['b']),
    }
    counts = _seg_counts(edge_blocks)

    by_dst = {'Customer': ['cc_near', 'sc_serves', 'cc_follows'],
              'Satellite': ['cs_near'],
              'RechargingStation': ['dr_near', 'sr_near', 'cr_near', 'rr_near']}
    for li in range(2):
        sums = _seg_sums(x, edge_blocks)
        sums['dr_near'] = counts['dr_near'][:, None] * x['Depot']
        cp = params['convs'][li]
        ln = params['ln'][li]
        newx = {}
        for dt, names in by_dst.items():
            nd = x[dt].shape[0]
            Ss = [sums[name][:nd] for name in names]
            cnts = [counts[name][:nd] for name in names]
            newx[dt] = _dense_layer(
                x[dt], Ss, cnts,
                [cp[n]['Wl'] for n in names], [cp[n]['bl'] for n in names],
                [cp[n]['Wr'] for n in names], ln[dt]['g'], ln[dt]['b'],
                br=1024 if dt == 'Customer' else None)
        x.update(newx)

    graph_vec = _attn_pool(x['Customer'], params['attn']['W'],
                           params['attn']['b'], NDC, 1024)
    return _head(graph_vec, scalars, params['scalar'], params['out'])
